# baseline (speedup 1.0000x reference)
import jax
import jax.numpy as jnp
from jax import lax
from jax.experimental import pallas as pl
from jax.experimental.pallas import tpu as pltpu
from jax.experimental.pallas import tpu_sc as plsc

N_NODES = 10000
NPAD = 10240
N_EDGES = 320000
NW = 32
EDGES_PER_W = N_EDGES // NW
CHUNK = 80
NCHUNK = EDGES_PER_W // CHUNK
ROWS_PER_TILE = NPAD // 16
BLK = 1000
NBLK = N_NODES // BLK

_SC_PARAMS = pltpu.CompilerParams(use_tc_tiling_on_sc=False,
                                  skip_device_barrier=True)


def _deg_body(dst_e, zeros_n, out, db0, db1, ones_v, acc, semd0, semd1):
    cid = lax.axis_index("c")
    sid = lax.axis_index("s")
    wid = sid * 2 + cid
    base = wid * EDGES_PER_W

    for i in range(CHUNK // 16):
        ones_v[pl.ds(i * 16, 16)] = jnp.ones((16,), jnp.float32)

    pltpu.sync_copy(zeros_n.at[pl.ds(sid * ROWS_PER_TILE, ROWS_PER_TILE)],
                    acc.at[pl.ds(sid * ROWS_PER_TILE, ROWS_PER_TILE)])
    plsc.subcore_barrier()

    pltpu.async_copy(dst_e.at[pl.ds(base, CHUNK)], db0, semd0)

    def pair(i, carry):
        c0 = 2 * i
        pltpu.async_copy(dst_e.at[pl.ds(base + (c0 + 1) * CHUNK, CHUNK)],
                         db1, semd1)
        pltpu.make_async_copy(dst_e.at[pl.ds(base, CHUNK)], db0, semd0).wait()
        pltpu.sync_copy(ones_v, acc.at[db0], add=True)
        pltpu.async_copy(dst_e.at[pl.ds(base + (c0 + 2) * CHUNK, CHUNK)],
                         db0, semd0)
        pltpu.make_async_copy(dst_e.at[pl.ds(base, CHUNK)], db1, semd1).wait()
        pltpu.sync_copy(ones_v, acc.at[db1], add=True)
        return carry

    lax.fori_loop(0, (NCHUNK - 1) // 2, pair, 0)
    pltpu.make_async_copy(dst_e.at[pl.ds(base, CHUNK)], db0, semd0).wait()
    pltpu.sync_copy(ones_v, acc.at[db0], add=True)
    plsc.subcore_barrier()

    pltpu.sync_copy(acc.at[pl.ds(sid * ROWS_PER_TILE, ROWS_PER_TILE)],
                    out.at[cid].at[pl.ds(sid * ROWS_PER_TILE, ROWS_PER_TILE)])


def _make_deg_kernel():
    return pl.kernel(
        _deg_body,
        out_type=jax.ShapeDtypeStruct((2, NPAD), jnp.float32),
        mesh=plsc.VectorSubcoreMesh(core_axis_name="c", subcore_axis_name="s"),
        compiler_params=_SC_PARAMS,
        scratch_types=[
            pltpu.VMEM((CHUNK,), jnp.int32),
            pltpu.VMEM((CHUNK,), jnp.int32),
            pltpu.VMEM((CHUNK,), jnp.float32),
            pltpu.VMEM_SHARED((NPAD,), jnp.float32),
            pltpu.SemaphoreType.DMA,
            pltpu.SemaphoreType.DMA,
        ],
    )


def _agg_body(src_e, dst_e, g, zeros, out, src_v, db0, db1, rows0, rows1, acc,
              sem0, sem1, semd0, semd1):
    cid = lax.axis_index("c")
    sid = lax.axis_index("s")
    wid = sid * 2 + cid
    base = wid * EDGES_PER_W
    rpt = ROWS_PER_TILE

    pltpu.sync_copy(zeros.at[pl.ds(sid * rpt, rpt)],
                    acc.at[pl.ds(sid * rpt, rpt)])
    pltpu.sync_copy(src_e.at[pl.ds(base, EDGES_PER_W)], src_v)
    plsc.subcore_barrier()

    pltpu.async_copy(g.at[src_v.at[pl.ds(0, CHUNK)]], rows0, sem0)
    pltpu.async_copy(dst_e.at[pl.ds(base, CHUNK)], db0, semd0)

    def pair(i, carry):
        c0 = 2 * i
        pltpu.async_copy(g.at[src_v.at[pl.ds((c0 + 1) * CHUNK, CHUNK)]],
                         rows1, sem1)
        pltpu.async_copy(dst_e.at[pl.ds(base + (c0 + 1) * CHUNK, CHUNK)],
                         db1, semd1)
        pltpu.make_async_copy(g.at[src_v.at[pl.ds(0, CHUNK)]],
                              rows0, sem0).wait()
        pltpu.make_async_copy(dst_e.at[pl.ds(base, CHUNK)], db0, semd0).wait()
        pltpu.sync_copy(rows0, acc.at[db0], add=True)
        pltpu.async_copy(g.at[src_v.at[pl.ds((c0 + 2) * CHUNK, CHUNK)]],
                         rows0, sem0)
        pltpu.async_copy(dst_e.at[pl.ds(base + (c0 + 2) * CHUNK, CHUNK)],
                         db0, semd0)
        pltpu.make_async_copy(g.at[src_v.at[pl.ds(0, CHUNK)]],
                              rows1, sem1).wait()
        pltpu.make_async_copy(dst_e.at[pl.ds(base, CHUNK)], db1, semd1).wait()
        pltpu.sync_copy(rows1, acc.at[db1], add=True)
        return carry

    lax.fori_loop(0, (NCHUNK - 1) // 2, pair, 0)
    pltpu.make_async_copy(g.at[src_v.at[pl.ds(0, CHUNK)]], rows0, sem0).wait()
    pltpu.make_async_copy(dst_e.at[pl.ds(base, CHUNK)], db0, semd0).wait()
    pltpu.sync_copy(rows0, acc.at[db0], add=True)
    plsc.subcore_barrier()

    pltpu.sync_copy(acc.at[pl.ds(sid * rpt, rpt)],
                    out.at[cid].at[pl.ds(sid * rpt, rpt)])


def _make_agg_kernel(feat):
    return pl.kernel(
        _agg_body,
        out_type=jax.ShapeDtypeStruct((2, NPAD, feat), jnp.float32),
        mesh=plsc.VectorSubcoreMesh(core_axis_name="c", subcore_axis_name="s"),
        compiler_params=_SC_PARAMS,
        scratch_types=[
            pltpu.VMEM((EDGES_PER_W,), jnp.int32),
            pltpu.VMEM((CHUNK,), jnp.int32),
            pltpu.VMEM((CHUNK,), jnp.int32),
            pltpu.VMEM((CHUNK, feat), jnp.float32),
            pltpu.VMEM((CHUNK, feat), jnp.float32),
            pltpu.VMEM_SHARED((NPAD, feat), jnp.float32),
            pltpu.SemaphoreType.DMA,
            pltpu.SemaphoreType.DMA,
            pltpu.SemaphoreType.DMA,
            pltpu.SemaphoreType.DMA,
        ],
    )


def _tc_a_body(x_ref, w1_ref, d0_ref, d1_ref, g1_ref, dinv_ref):
    dinv = lax.rsqrt(d0_ref[...] + d1_ref[...] + 1.0)
    h = jnp.dot(x_ref[...], w1_ref[...], preferred_element_type=jnp.float32)
    g1_ref[...] = h * dinv
    dinv_ref[...] = dinv


def _tc_a(x, w1, d0, d1):
    return pl.pallas_call(
        _tc_a_body,
        grid=(NBLK,),
        in_specs=[
            pl.BlockSpec((BLK, 128), lambda i: (i, 0)),
            pl.BlockSpec((128, 32), lambda i: (0, 0)),
            pl.BlockSpec((BLK, 1), lambda i: (i, 0)),
            pl.BlockSpec((BLK, 1), lambda i: (i, 0)),
        ],
        out_specs=[
            pl.BlockSpec((BLK, 32), lambda i: (i, 0)),
            pl.BlockSpec((BLK, 1), lambda i: (i, 0)),
        ],
        out_shape=[
            jax.ShapeDtypeStruct((N_NODES, 32), jnp.float32),
            jax.ShapeDtypeStruct((N_NODES, 1), jnp.float32),
        ],
    )(x, w1, d0, d1)


def _tc_b_body(a0_ref, a1_ref, g1_ref, dinv_ref, b1_ref, w2_ref, g2_ref):
    dinv = dinv_ref[...]
    o1 = ((a0_ref[...] + a1_ref[...] + g1_ref[...]) * dinv + b1_ref[...])
    o1 = jnp.maximum(o1, 0.0)
    h2 = jnp.dot(o1, w2_ref[...], preferred_element_type=jnp.float32)
    g2_ref[...] = h2 * dinv


def _tc_b(a0, a1, g1, dinv, b1, w2):
    return pl.pallas_call(
        _tc_b_body,
        grid=(NBLK,),
        in_specs=[
            pl.BlockSpec((BLK, 32), lambda i: (i, 0)),
            pl.BlockSpec((BLK, 32), lambda i: (i, 0)),
            pl.BlockSpec((BLK, 32), lambda i: (i, 0)),
            pl.BlockSpec((BLK, 1), lambda i: (i, 0)),
            pl.BlockSpec((1, 32), lambda i: (0, 0)),
            pl.BlockSpec((32, 64), lambda i: (0, 0)),
        ],
        out_specs=pl.BlockSpec((BLK, 64), lambda i: (i, 0)),
        out_shape=jax.ShapeDtypeStruct((N_NODES, 64), jnp.float32),
    )(a0, a1, g1, dinv, b1, w2)


def _tc_c_body(a0_ref, a1_ref, g2_ref, dinv_ref, b2_ref, out_ref):
    out_ref[...] = ((a0_ref[...] + a1_ref[...] + g2_ref[...])
                    * dinv_ref[...] + b2_ref[...])


def _tc_c(a0, a1, g2, dinv, b2):
    return pl.pallas_call(
        _tc_c_body,
        grid=(NBLK,),
        in_specs=[
            pl.BlockSpec((BLK, 64), lambda i: (i, 0)),
            pl.BlockSpec((BLK, 64), lambda i: (i, 0)),
            pl.BlockSpec((BLK, 64), lambda i: (i, 0)),
            pl.BlockSpec((BLK, 1), lambda i: (i, 0)),
            pl.BlockSpec((1, 64), lambda i: (0, 0)),
        ],
        out_specs=pl.BlockSpec((BLK, 64), lambda i: (i, 0)),
        out_shape=jax.ShapeDtypeStruct((N_NODES, 64), jnp.float32),
    )(a0, a1, g2, dinv, b2)


@jax.jit
def kernel(x, edge_index, W1, b1, W2, b2):
    src_e = edge_index[0].astype(jnp.int32)
    dst_e = edge_index[1].astype(jnp.int32)

    zeros_n = jnp.zeros((NPAD,), jnp.float32)
    zeros32 = jnp.zeros((NPAD, 32), jnp.float32)
    zeros64 = jnp.zeros((NPAD, 64), jnp.float32)

    deg = _make_deg_kernel()(dst_e, zeros_n)
    d0 = deg[0, :N_NODES].reshape(N_NODES, 1)
    d1 = deg[1, :N_NODES].reshape(N_NODES, 1)

    g1, dinv = _tc_a(x, W1, d0, d1)

    agg1 = _make_agg_kernel(32)(src_e, dst_e, g1, zeros32)
    g2 = _tc_b(agg1[0, :N_NODES], agg1[1, :N_NODES], g1, dinv,
               b1.reshape(1, 32), W2)

    agg2 = _make_agg_kernel(64)(src_e, dst_e, g2, zeros64)
    return _tc_c(agg2[0, :N_NODES], agg2[1, :N_NODES], g2, dinv,
                 b2.reshape(1, 64))


# speedup vs baseline: 1.0709x; 1.0709x over previous
import jax
import jax.numpy as jnp
from jax import lax
from jax.experimental import pallas as pl
from jax.experimental.pallas import tpu as pltpu
from jax.experimental.pallas import tpu_sc as plsc

N_NODES = 10000
NPAD = 10240
N_EDGES = 320000
NW = 32
EDGES_PER_W = N_EDGES // NW
CHUNK = 80
NCHUNK = EDGES_PER_W // CHUNK
ROWS_PER_TILE = NPAD // 16
BLK = 1000
NBLK = N_NODES // BLK

_SC_PARAMS = pltpu.CompilerParams(use_tc_tiling_on_sc=False,
                                  skip_device_barrier=True)


def _deg_body(dst3, zeros_n, out, dst_v, ones_v, acc, sem):
    cid = lax.axis_index("c")
    sid = lax.axis_index("s")
    wid = sid * 2 + cid

    for i in range(CHUNK // 16):
        ones_v[pl.ds(i * 16, 16)] = jnp.ones((16,), jnp.float32)

    pltpu.sync_copy(zeros_n.at[pl.ds(sid * ROWS_PER_TILE, ROWS_PER_TILE)],
                    acc.at[pl.ds(sid * ROWS_PER_TILE, ROWS_PER_TILE)])
    pltpu.sync_copy(dst3.at[wid], dst_v)
    plsc.subcore_barrier()

    def chunk(c, carry):
        pltpu.sync_copy(ones_v, acc.at[dst_v.at[c]], add=True)
        return carry

    lax.fori_loop(0, NCHUNK, chunk, 0)
    plsc.subcore_barrier()

    pltpu.sync_copy(acc.at[pl.ds(sid * ROWS_PER_TILE, ROWS_PER_TILE)],
                    out.at[cid].at[pl.ds(sid * ROWS_PER_TILE, ROWS_PER_TILE)])


def _make_deg_kernel():
    return pl.kernel(
        _deg_body,
        out_type=jax.ShapeDtypeStruct((2, NPAD), jnp.float32),
        mesh=plsc.VectorSubcoreMesh(core_axis_name="c", subcore_axis_name="s"),
        compiler_params=_SC_PARAMS,
        scratch_types=[
            pltpu.VMEM((NCHUNK, CHUNK), jnp.int32),
            pltpu.VMEM((CHUNK,), jnp.float32),
            pltpu.VMEM_SHARED((NPAD,), jnp.float32),
            pltpu.SemaphoreType.DMA,
        ],
    )


def _agg_body(src3, dst4, g, zeros, out, src_v, db0, db1, rows0, rows1, acc,
              sem0, sem1, semd0, semd1):
    cid = lax.axis_index("c")
    sid = lax.axis_index("s")
    wid = sid * 2 + cid
    rpt = ROWS_PER_TILE
    dst2 = dst4.at[wid]

    pltpu.sync_copy(zeros.at[pl.ds(sid * rpt, rpt)],
                    acc.at[pl.ds(sid * rpt, rpt)])
    pltpu.sync_copy(src3.at[wid], src_v)
    plsc.subcore_barrier()

    pltpu.async_copy(g.at[src_v.at[0]], rows0, sem0)
    pltpu.async_copy(dst2.at[0], db0, semd0)

    def pair(i, carry):
        c0 = 2 * i
        pltpu.async_copy(g.at[src_v.at[c0 + 1]], rows1, sem1)
        pltpu.async_copy(dst2.at[c0 + 1], db1, semd1)
        pltpu.make_async_copy(g.at[src_v.at[c0]], rows0, sem0).wait()
        pltpu.make_async_copy(dst2.at[c0], db0, semd0).wait()
        pltpu.sync_copy(rows0, acc.at[db0.at[0]], add=True)
        pltpu.async_copy(g.at[src_v.at[c0 + 2]], rows0, sem0)
        pltpu.async_copy(dst2.at[c0 + 2], db0, semd0)
        pltpu.make_async_copy(g.at[src_v.at[c0 + 1]], rows1, sem1).wait()
        pltpu.make_async_copy(dst2.at[c0 + 1], db1, semd1).wait()
        pltpu.sync_copy(rows1, acc.at[db1.at[0]], add=True)
        return carry

    lax.fori_loop(0, (NCHUNK - 1) // 2, pair, 0)
    pltpu.make_async_copy(g.at[src_v.at[NCHUNK - 1]], rows0, sem0).wait()
    pltpu.make_async_copy(dst2.at[NCHUNK - 1], db0, semd0).wait()
    pltpu.sync_copy(rows0, acc.at[db0.at[0]], add=True)
    plsc.subcore_barrier()

    pltpu.sync_copy(acc.at[pl.ds(sid * rpt, rpt)],
                    out.at[cid].at[pl.ds(sid * rpt, rpt)])


def _make_agg_kernel(feat):
    return pl.kernel(
        _agg_body,
        out_type=jax.ShapeDtypeStruct((2, NPAD, feat), jnp.float32),
        mesh=plsc.VectorSubcoreMesh(core_axis_name="c", subcore_axis_name="s"),
        compiler_params=_SC_PARAMS,
        scratch_types=[
            pltpu.VMEM((NCHUNK, CHUNK), jnp.int32),
            pltpu.VMEM((1, CHUNK), jnp.int32),
            pltpu.VMEM((1, CHUNK), jnp.int32),
            pltpu.VMEM((CHUNK, feat), jnp.float32),
            pltpu.VMEM((CHUNK, feat), jnp.float32),
            pltpu.VMEM_SHARED((NPAD, feat), jnp.float32),
            pltpu.SemaphoreType.DMA,
            pltpu.SemaphoreType.DMA,
            pltpu.SemaphoreType.DMA,
            pltpu.SemaphoreType.DMA,
        ],
    )


def _tc_a_body(x_ref, w1_ref, d0_ref, d1_ref, g1_ref, dinv_ref):
    dinv = lax.rsqrt(d0_ref[...] + d1_ref[...] + 1.0)
    h = jnp.dot(x_ref[...], w1_ref[...], preferred_element_type=jnp.float32)
    g1_ref[...] = h * dinv
    dinv_ref[...] = dinv


def _tc_a(x, w1, d0, d1):
    return pl.pallas_call(
        _tc_a_body,
        grid=(NBLK,),
        in_specs=[
            pl.BlockSpec((BLK, 128), lambda i: (i, 0)),
            pl.BlockSpec((128, 32), lambda i: (0, 0)),
            pl.BlockSpec((BLK, 1), lambda i: (i, 0)),
            pl.BlockSpec((BLK, 1), lambda i: (i, 0)),
        ],
        out_specs=[
            pl.BlockSpec((BLK, 32), lambda i: (i, 0)),
            pl.BlockSpec((BLK, 1), lambda i: (i, 0)),
        ],
        out_shape=[
            jax.ShapeDtypeStruct((N_NODES, 32), jnp.float32),
            jax.ShapeDtypeStruct((N_NODES, 1), jnp.float32),
        ],
    )(x, w1, d0, d1)


def _tc_b_body(a0_ref, a1_ref, g1_ref, dinv_ref, b1_ref, w2_ref, g2_ref):
    dinv = dinv_ref[...]
    o1 = ((a0_ref[...] + a1_ref[...] + g1_ref[...]) * dinv + b1_ref[...])
    o1 = jnp.maximum(o1, 0.0)
    h2 = jnp.dot(o1, w2_ref[...], preferred_element_type=jnp.float32)
    g2_ref[...] = h2 * dinv


def _tc_b(a0, a1, g1, dinv, b1, w2):
    return pl.pallas_call(
        _tc_b_body,
        grid=(NBLK,),
        in_specs=[
            pl.BlockSpec((BLK, 32), lambda i: (i, 0)),
            pl.BlockSpec((BLK, 32), lambda i: (i, 0)),
            pl.BlockSpec((BLK, 32), lambda i: (i, 0)),
            pl.BlockSpec((BLK, 1), lambda i: (i, 0)),
            pl.BlockSpec((1, 32), lambda i: (0, 0)),
            pl.BlockSpec((32, 64), lambda i: (0, 0)),
        ],
        out_specs=pl.BlockSpec((BLK, 64), lambda i: (i, 0)),
        out_shape=jax.ShapeDtypeStruct((N_NODES, 64), jnp.float32),
    )(a0, a1, g1, dinv, b1, w2)


def _tc_c_body(a0_ref, a1_ref, g2_ref, dinv_ref, b2_ref, out_ref):
    out_ref[...] = ((a0_ref[...] + a1_ref[...] + g2_ref[...])
                    * dinv_ref[...] + b2_ref[...])


def _tc_c(a0, a1, g2, dinv, b2):
    return pl.pallas_call(
        _tc_c_body,
        grid=(NBLK,),
        in_specs=[
            pl.BlockSpec((BLK, 64), lambda i: (i, 0)),
            pl.BlockSpec((BLK, 64), lambda i: (i, 0)),
            pl.BlockSpec((BLK, 64), lambda i: (i, 0)),
            pl.BlockSpec((BLK, 1), lambda i: (i, 0)),
            pl.BlockSpec((1, 64), lambda i: (0, 0)),
        ],
        out_specs=pl.BlockSpec((BLK, 64), lambda i: (i, 0)),
        out_shape=jax.ShapeDtypeStruct((N_NODES, 64), jnp.float32),
    )(a0, a1, g2, dinv, b2)


@jax.jit
def kernel(x, edge_index, W1, b1, W2, b2):
    src_e = edge_index[0].astype(jnp.int32)
    dst_e = edge_index[1].astype(jnp.int32)
    src3 = src_e.reshape(NW, NCHUNK, CHUNK)
    dst3 = dst_e.reshape(NW, NCHUNK, CHUNK)
    dst4 = dst_e.reshape(NW, NCHUNK, 1, CHUNK)

    zeros_n = jnp.zeros((NPAD,), jnp.float32)
    zeros32 = jnp.zeros((NPAD, 32), jnp.float32)
    zeros64 = jnp.zeros((NPAD, 64), jnp.float32)

    deg = _make_deg_kernel()(dst3, zeros_n)
    d0 = deg[0, :N_NODES].reshape(N_NODES, 1)
    d1 = deg[1, :N_NODES].reshape(N_NODES, 1)

    g1, dinv = _tc_a(x, W1, d0, d1)

    agg1 = _make_agg_kernel(32)(src3, dst4, g1, zeros32)
    g2 = _tc_b(agg1[0], agg1[1], g1, dinv, b1.reshape(1, 32), W2)

    agg2 = _make_agg_kernel(64)(src3, dst4, g2, zeros64)
    return _tc_c(agg2[0], agg2[1], g2, dinv, b2.reshape(1, 64))


# speedup vs baseline: 1.0775x; 1.0061x over previous
import jax
import jax.numpy as jnp
from jax import lax
from jax.experimental import pallas as pl
from jax.experimental.pallas import tpu as pltpu
from jax.experimental.pallas import tpu_sc as plsc

N_NODES = 10000
NPAD = 10240
N_EDGES = 320000
NW = 32
EDGES_PER_W = N_EDGES // NW
CHUNK = 80
NCHUNK = EDGES_PER_W // CHUNK
ROWS_PER_TILE = NPAD // 16
BLK = 1000
NBLK = N_NODES // BLK
BLKA = 2000
NBLKA = N_NODES // BLKA

_SC_PARAMS = pltpu.CompilerParams(use_tc_tiling_on_sc=False,
                                  skip_device_barrier=True)


def _deg_body(dst4, zeros_n, out, dst_v, ones_v, acc, sem):
    cid = lax.axis_index("c")
    sid = lax.axis_index("s")
    wid = sid * 2 + cid

    for i in range(CHUNK // 16):
        ones_v[pl.ds(i * 16, 16)] = jnp.ones((16,), jnp.float32)

    pltpu.sync_copy(zeros_n.at[pl.ds(sid * ROWS_PER_TILE, ROWS_PER_TILE)],
                    acc.at[pl.ds(sid * ROWS_PER_TILE, ROWS_PER_TILE)])
    pltpu.sync_copy(dst4.at[wid], dst_v)
    plsc.subcore_barrier()

    def chunk(c, carry):
        pltpu.sync_copy(ones_v, acc.at[dst_v.at[c].at[0]], add=True)
        return carry

    lax.fori_loop(0, NCHUNK, chunk, 0)
    plsc.subcore_barrier()

    pltpu.sync_copy(acc.at[pl.ds(sid * ROWS_PER_TILE, ROWS_PER_TILE)],
                    out.at[cid].at[pl.ds(sid * ROWS_PER_TILE, ROWS_PER_TILE)])


def _make_deg_kernel():
    return pl.kernel(
        _deg_body,
        out_type=jax.ShapeDtypeStruct((2, NPAD), jnp.float32),
        mesh=plsc.VectorSubcoreMesh(core_axis_name="c", subcore_axis_name="s"),
        compiler_params=_SC_PARAMS,
        scratch_types=[
            pltpu.VMEM((NCHUNK, 1, CHUNK), jnp.int32),
            pltpu.VMEM((CHUNK,), jnp.float32),
            pltpu.VMEM_SHARED((NPAD,), jnp.float32),
            pltpu.SemaphoreType.DMA,
        ],
    )


def _agg_body(src3, dst4, g, zeros, out, src_v, db0, db1, rows0, rows1, acc,
              sem0, sem1, semd0, semd1):
    cid = lax.axis_index("c")
    sid = lax.axis_index("s")
    wid = sid * 2 + cid
    rpt = ROWS_PER_TILE
    dst2 = dst4.at[wid]

    pltpu.sync_copy(zeros.at[pl.ds(sid * rpt, rpt)],
                    acc.at[pl.ds(sid * rpt, rpt)])
    pltpu.sync_copy(src3.at[wid], src_v)
    plsc.subcore_barrier()

    pltpu.async_copy(g.at[src_v.at[0]], rows0, sem0)
    pltpu.async_copy(dst2.at[0], db0, semd0)

    def pair(i, carry):
        c0 = 2 * i
        pltpu.async_copy(g.at[src_v.at[c0 + 1]], rows1, sem1)
        pltpu.async_copy(dst2.at[c0 + 1], db1, semd1)
        pltpu.make_async_copy(g.at[src_v.at[c0]], rows0, sem0).wait()
        pltpu.make_async_copy(dst2.at[c0], db0, semd0).wait()
        pltpu.sync_copy(rows0, acc.at[db0.at[0]], add=True)
        pltpu.async_copy(g.at[src_v.at[c0 + 2]], rows0, sem0)
        pltpu.async_copy(dst2.at[c0 + 2], db0, semd0)
        pltpu.make_async_copy(g.at[src_v.at[c0 + 1]], rows1, sem1).wait()
        pltpu.make_async_copy(dst2.at[c0 + 1], db1, semd1).wait()
        pltpu.sync_copy(rows1, acc.at[db1.at[0]], add=True)
        return carry

    lax.fori_loop(0, (NCHUNK - 1) // 2, pair, 0)
    pltpu.make_async_copy(g.at[src_v.at[NCHUNK - 1]], rows0, sem0).wait()
    pltpu.make_async_copy(dst2.at[NCHUNK - 1], db0, semd0).wait()
    pltpu.sync_copy(rows0, acc.at[db0.at[0]], add=True)
    plsc.subcore_barrier()

    pltpu.sync_copy(acc.at[pl.ds(sid * rpt, rpt)],
                    out.at[cid].at[pl.ds(sid * rpt, rpt)])


def _make_agg_kernel(feat):
    return pl.kernel(
        _agg_body,
        out_type=jax.ShapeDtypeStruct((2, NPAD, feat), jnp.float32),
        mesh=plsc.VectorSubcoreMesh(core_axis_name="c", subcore_axis_name="s"),
        compiler_params=_SC_PARAMS,
        scratch_types=[
            pltpu.VMEM((NCHUNK, CHUNK), jnp.int32),
            pltpu.VMEM((1, CHUNK), jnp.int32),
            pltpu.VMEM((1, CHUNK), jnp.int32),
            pltpu.VMEM((CHUNK, feat), jnp.float32),
            pltpu.VMEM((CHUNK, feat), jnp.float32),
            pltpu.VMEM_SHARED((NPAD, feat), jnp.float32),
            pltpu.SemaphoreType.DMA,
            pltpu.SemaphoreType.DMA,
            pltpu.SemaphoreType.DMA,
            pltpu.SemaphoreType.DMA,
        ],
    )


def _tc_a_body(x_ref, w1_ref, d0_ref, d1_ref, g1_ref, dinv_ref):
    dinv = lax.rsqrt(d0_ref[...] + d1_ref[...] + 1.0)
    h = jnp.dot(x_ref[...], w1_ref[...], preferred_element_type=jnp.float32)
    g1_ref[...] = h * dinv
    dinv_ref[...] = dinv


def _tc_a(x, w1, d0, d1):
    return pl.pallas_call(
        _tc_a_body,
        grid=(NBLKA,),
        in_specs=[
            pl.BlockSpec((BLKA, 128), lambda i: (i, 0)),
            pl.BlockSpec((128, 32), lambda i: (0, 0)),
            pl.BlockSpec((BLKA, 1), lambda i: (i, 0)),
            pl.BlockSpec((BLKA, 1), lambda i: (i, 0)),
        ],
        out_specs=[
            pl.BlockSpec((BLKA, 32), lambda i: (i, 0)),
            pl.BlockSpec((BLKA, 1), lambda i: (i, 0)),
        ],
        out_shape=[
            jax.ShapeDtypeStruct((N_NODES, 32), jnp.float32),
            jax.ShapeDtypeStruct((N_NODES, 1), jnp.float32),
        ],
    )(x, w1, d0, d1)


def _tc_b_body(a0_ref, a1_ref, g1_ref, dinv_ref, b1_ref, w2_ref, g2_ref):
    dinv = dinv_ref[...]
    o1 = ((a0_ref[...] + a1_ref[...] + g1_ref[...]) * dinv + b1_ref[...])
    o1 = jnp.maximum(o1, 0.0)
    h2 = jnp.dot(o1, w2_ref[...], preferred_element_type=jnp.float32)
    g2_ref[...] = h2 * dinv


def _tc_b(a0, a1, g1, dinv, b1, w2):
    return pl.pallas_call(
        _tc_b_body,
        grid=(NBLK,),
        in_specs=[
            pl.BlockSpec((BLK, 32), lambda i: (i, 0)),
            pl.BlockSpec((BLK, 32), lambda i: (i, 0)),
            pl.BlockSpec((BLK, 32), lambda i: (i, 0)),
            pl.BlockSpec((BLK, 1), lambda i: (i, 0)),
            pl.BlockSpec((1, 32), lambda i: (0, 0)),
            pl.BlockSpec((32, 64), lambda i: (0, 0)),
        ],
        out_specs=pl.BlockSpec((BLK, 64), lambda i: (i, 0)),
        out_shape=jax.ShapeDtypeStruct((N_NODES, 64), jnp.float32),
    )(a0, a1, g1, dinv, b1, w2)


def _tc_c_body(a0_ref, a1_ref, g2_ref, dinv_ref, b2_ref, out_ref):
    out_ref[...] = ((a0_ref[...] + a1_ref[...] + g2_ref[...])
                    * dinv_ref[...] + b2_ref[...])


def _tc_c(a0, a1, g2, dinv, b2):
    return pl.pallas_call(
        _tc_c_body,
        grid=(NBLK,),
        in_specs=[
            pl.BlockSpec((BLK, 64), lambda i: (i, 0)),
            pl.BlockSpec((BLK, 64), lambda i: (i, 0)),
            pl.BlockSpec((BLK, 64), lambda i: (i, 0)),
            pl.BlockSpec((BLK, 1), lambda i: (i, 0)),
            pl.BlockSpec((1, 64), lambda i: (0, 0)),
        ],
        out_specs=pl.BlockSpec((BLK, 64), lambda i: (i, 0)),
        out_shape=jax.ShapeDtypeStruct((N_NODES, 64), jnp.float32),
    )(a0, a1, g2, dinv, b2)


@jax.jit
def kernel(x, edge_index, W1, b1, W2, b2):
    src_e = edge_index[0].astype(jnp.int32)
    dst_e = edge_index[1].astype(jnp.int32)
    src3 = src_e.reshape(NW, NCHUNK, CHUNK)
    dst4 = dst_e.reshape(NW, NCHUNK, 1, CHUNK)

    zeros_n = jnp.zeros((NPAD,), jnp.float32)
    zeros32 = jnp.zeros((NPAD, 32), jnp.float32)
    zeros64 = jnp.zeros((NPAD, 64), jnp.float32)

    deg = _make_deg_kernel()(dst4, zeros_n)
    d0 = deg[0, :N_NODES].reshape(N_NODES, 1)
    d1 = deg[1, :N_NODES].reshape(N_NODES, 1)

    g1, dinv = _tc_a(x, W1, d0, d1)

    agg1 = _make_agg_kernel(32)(src3, dst4, g1, zeros32)
    g2 = _tc_b(agg1[0], agg1[1], g1, dinv, b1.reshape(1, 32), W2)

    agg2 = _make_agg_kernel(64)(src3, dst4, g2, zeros64)
    return _tc_c(agg2[0], agg2[1], g2, dinv, b2.reshape(1, 64))


# speedup vs baseline: 1.0776x; 1.0002x over previous
import jax
import jax.numpy as jnp
from jax import lax
from jax.experimental import pallas as pl
from jax.experimental.pallas import tpu as pltpu
from jax.experimental.pallas import tpu_sc as plsc

N_NODES = 10000
NPAD = 10240
N_EDGES = 320000
NW = 32
EDGES_PER_W = N_EDGES // NW
CHUNK = 80
NCHUNK = EDGES_PER_W // CHUNK
ROWS_PER_TILE = NPAD // 16
BLK = 1000
NBLK = N_NODES // BLK
BLKA = 2000
NBLKA = N_NODES // BLKA

_SC_PARAMS = pltpu.CompilerParams(use_tc_tiling_on_sc=False,
                                  skip_device_barrier=True)


def _deg_body(dst4, zeros_n, out, dst_v, ones_v, acc, sem):
    cid = lax.axis_index("c")
    sid = lax.axis_index("s")
    wid = sid * 2 + cid

    for i in range(CHUNK // 16):
        ones_v[pl.ds(i * 16, 16)] = jnp.ones((16,), jnp.float32)

    pltpu.sync_copy(zeros_n.at[pl.ds(sid * ROWS_PER_TILE, ROWS_PER_TILE)],
                    acc.at[pl.ds(sid * ROWS_PER_TILE, ROWS_PER_TILE)])
    pltpu.sync_copy(dst4.at[wid], dst_v)
    plsc.subcore_barrier()

    def chunk(c, carry):
        pltpu.sync_copy(ones_v, acc.at[dst_v.at[c].at[0]], add=True)
        return carry

    lax.fori_loop(0, NCHUNK, chunk, 0)
    plsc.subcore_barrier()

    pltpu.sync_copy(acc.at[pl.ds(sid * ROWS_PER_TILE, ROWS_PER_TILE)],
                    out.at[cid].at[pl.ds(sid * ROWS_PER_TILE, ROWS_PER_TILE)])


def _make_deg_kernel():
    return pl.kernel(
        _deg_body,
        out_type=jax.ShapeDtypeStruct((2, NPAD), jnp.float32),
        mesh=plsc.VectorSubcoreMesh(core_axis_name="c", subcore_axis_name="s"),
        compiler_params=_SC_PARAMS,
        scratch_types=[
            pltpu.VMEM((NCHUNK, 1, CHUNK), jnp.int32),
            pltpu.VMEM((CHUNK,), jnp.float32),
            pltpu.VMEM_SHARED((NPAD,), jnp.float32),
            pltpu.SemaphoreType.DMA,
        ],
    )


def _agg_body(src3, dst4, g, zeros, out, src_v, db0, db1, rows0, rows1, acc,
              sem0, sem1, semd0, semd1):
    cid = lax.axis_index("c")
    sid = lax.axis_index("s")
    wid = sid * 2 + cid
    rpt = ROWS_PER_TILE
    dst2 = dst4.at[wid]

    pltpu.sync_copy(zeros.at[pl.ds(sid * rpt, rpt)],
                    acc.at[pl.ds(sid * rpt, rpt)])
    pltpu.sync_copy(src3.at[wid], src_v)
    plsc.subcore_barrier()

    pltpu.async_copy(g.at[src_v.at[0]], rows0, sem0)
    pltpu.async_copy(dst2.at[0], db0, semd0)

    def pair(i, carry):
        c0 = 2 * i
        pltpu.async_copy(g.at[src_v.at[c0 + 1]], rows1, sem1)
        pltpu.async_copy(dst2.at[c0 + 1], db1, semd1)
        pltpu.make_async_copy(g.at[src_v.at[c0]], rows0, sem0).wait()
        pltpu.make_async_copy(dst2.at[c0], db0, semd0).wait()
        pltpu.sync_copy(rows0, acc.at[db0.at[0]], add=True)
        pltpu.async_copy(g.at[src_v.at[c0 + 2]], rows0, sem0)
        pltpu.async_copy(dst2.at[c0 + 2], db0, semd0)
        pltpu.make_async_copy(g.at[src_v.at[c0 + 1]], rows1, sem1).wait()
        pltpu.make_async_copy(dst2.at[c0 + 1], db1, semd1).wait()
        pltpu.sync_copy(rows1, acc.at[db1.at[0]], add=True)
        return carry

    lax.fori_loop(0, (NCHUNK - 1) // 2, pair, 0)
    pltpu.make_async_copy(g.at[src_v.at[NCHUNK - 1]], rows0, sem0).wait()
    pltpu.make_async_copy(dst2.at[NCHUNK - 1], db0, semd0).wait()
    pltpu.sync_copy(rows0, acc.at[db0.at[0]], add=True)
    plsc.subcore_barrier()

    pltpu.sync_copy(acc.at[pl.ds(sid * rpt, rpt)],
                    out.at[cid].at[pl.ds(sid * rpt, rpt)])


def _make_agg_kernel(feat):
    return pl.kernel(
        _agg_body,
        out_type=jax.ShapeDtypeStruct((2, NPAD, feat), jnp.float32),
        mesh=plsc.VectorSubcoreMesh(core_axis_name="c", subcore_axis_name="s"),
        compiler_params=_SC_PARAMS,
        scratch_types=[
            pltpu.VMEM((NCHUNK, CHUNK), jnp.int32),
            pltpu.VMEM((1, CHUNK), jnp.int32),
            pltpu.VMEM((1, CHUNK), jnp.int32),
            pltpu.VMEM((CHUNK, feat), jnp.float32),
            pltpu.VMEM((CHUNK, feat), jnp.float32),
            pltpu.VMEM_SHARED((NPAD, feat), jnp.float32),
            pltpu.SemaphoreType.DMA,
            pltpu.SemaphoreType.DMA,
            pltpu.SemaphoreType.DMA,
            pltpu.SemaphoreType.DMA,
        ],
    )


def _tc_h1_body(x_ref, w1_ref, h_ref):
    h_ref[...] = jnp.dot(x_ref[...], w1_ref[...],
                         preferred_element_type=jnp.float32)


def _tc_h1(x, w1):
    return pl.pallas_call(
        _tc_h1_body,
        grid=(NBLKA,),
        in_specs=[
            pl.BlockSpec((BLKA, 128), lambda i: (i, 0)),
            pl.BlockSpec((128, 32), lambda i: (0, 0)),
        ],
        out_specs=pl.BlockSpec((BLKA, 32), lambda i: (i, 0)),
        out_shape=jax.ShapeDtypeStruct((N_NODES, 32), jnp.float32),
    )(x, w1)


def _tc_a_body(h_ref, d0_ref, d1_ref, g1_ref, dinv_ref):
    dinv = lax.rsqrt(d0_ref[...] + d1_ref[...] + 1.0)
    g1_ref[...] = h_ref[...] * dinv
    dinv_ref[...] = dinv


def _tc_a(h, d0, d1):
    return pl.pallas_call(
        _tc_a_body,
        grid=(NBLKA,),
        in_specs=[
            pl.BlockSpec((BLKA, 32), lambda i: (i, 0)),
            pl.BlockSpec((BLKA, 1), lambda i: (i, 0)),
            pl.BlockSpec((BLKA, 1), lambda i: (i, 0)),
        ],
        out_specs=[
            pl.BlockSpec((BLKA, 32), lambda i: (i, 0)),
            pl.BlockSpec((BLKA, 1), lambda i: (i, 0)),
        ],
        out_shape=[
            jax.ShapeDtypeStruct((N_NODES, 32), jnp.float32),
            jax.ShapeDtypeStruct((N_NODES, 1), jnp.float32),
        ],
    )(h, d0, d1)


def _tc_b_body(a0_ref, a1_ref, g1_ref, dinv_ref, b1_ref, w2_ref, g2_ref):
    dinv = dinv_ref[...]
    o1 = ((a0_ref[...] + a1_ref[...] + g1_ref[...]) * dinv + b1_ref[...])
    o1 = jnp.maximum(o1, 0.0)
    h2 = jnp.dot(o1, w2_ref[...], preferred_element_type=jnp.float32)
    g2_ref[...] = h2 * dinv


def _tc_b(a0, a1, g1, dinv, b1, w2):
    return pl.pallas_call(
        _tc_b_body,
        grid=(NBLK,),
        in_specs=[
            pl.BlockSpec((BLK, 32), lambda i: (i, 0)),
            pl.BlockSpec((BLK, 32), lambda i: (i, 0)),
            pl.BlockSpec((BLK, 32), lambda i: (i, 0)),
            pl.BlockSpec((BLK, 1), lambda i: (i, 0)),
            pl.BlockSpec((1, 32), lambda i: (0, 0)),
            pl.BlockSpec((32, 64), lambda i: (0, 0)),
        ],
        out_specs=pl.BlockSpec((BLK, 64), lambda i: (i, 0)),
        out_shape=jax.ShapeDtypeStruct((N_NODES, 64), jnp.float32),
    )(a0, a1, g1, dinv, b1, w2)


def _tc_c_body(a0_ref, a1_ref, g2_ref, dinv_ref, b2_ref, out_ref):
    out_ref[...] = ((a0_ref[...] + a1_ref[...] + g2_ref[...])
                    * dinv_ref[...] + b2_ref[...])


def _tc_c(a0, a1, g2, dinv, b2):
    return pl.pallas_call(
        _tc_c_body,
        grid=(NBLK,),
        in_specs=[
            pl.BlockSpec((BLK, 64), lambda i: (i, 0)),
            pl.BlockSpec((BLK, 64), lambda i: (i, 0)),
            pl.BlockSpec((BLK, 64), lambda i: (i, 0)),
            pl.BlockSpec((BLK, 1), lambda i: (i, 0)),
            pl.BlockSpec((1, 64), lambda i: (0, 0)),
        ],
        out_specs=pl.BlockSpec((BLK, 64), lambda i: (i, 0)),
        out_shape=jax.ShapeDtypeStruct((N_NODES, 64), jnp.float32),
    )(a0, a1, g2, dinv, b2)


@jax.jit
def kernel(x, edge_index, W1, b1, W2, b2):
    src_e = edge_index[0].astype(jnp.int32)
    dst_e = edge_index[1].astype(jnp.int32)
    src3 = src_e.reshape(NW, NCHUNK, CHUNK)
    dst4 = dst_e.reshape(NW, NCHUNK, 1, CHUNK)

    zeros_n = jnp.zeros((NPAD,), jnp.float32)
    zeros32 = jnp.zeros((NPAD, 32), jnp.float32)
    zeros64 = jnp.zeros((NPAD, 64), jnp.float32)

    h1 = _tc_h1(x, W1)
    deg = _make_deg_kernel()(dst4, zeros_n)
    d0 = deg[0, :N_NODES].reshape(N_NODES, 1)
    d1 = deg[1, :N_NODES].reshape(N_NODES, 1)

    g1, dinv = _tc_a(h1, d0, d1)

    agg1 = _make_agg_kernel(32)(src3, dst4, g1, zeros32)
    g2 = _tc_b(agg1[0], agg1[1], g1, dinv, b1.reshape(1, 32), W2)

    agg2 = _make_agg_kernel(64)(src3, dst4, g2, zeros64)
    return _tc_c(agg2[0], agg2[1], g2, dinv, b2.reshape(1, 64))


# speedup vs baseline: 1.0782x; 1.0005x over previous
import jax
import jax.numpy as jnp
from jax import lax
from jax.experimental import pallas as pl
from jax.experimental.pallas import tpu as pltpu
from jax.experimental.pallas import tpu_sc as plsc

N_NODES = 10000
NPAD = 10240
N_EDGES = 320000
NW = 32
EDGES_PER_W = N_EDGES // NW
CHUNK = 80
NCHUNK = EDGES_PER_W // CHUNK
ROWS_PER_TILE = NPAD // 16
BLK = 1000
NBLK = N_NODES // BLK
BLKA = 2000
NBLKA = N_NODES // BLKA

_SC_PARAMS = pltpu.CompilerParams(use_tc_tiling_on_sc=False,
                                  skip_device_barrier=True)


def _deg_body(dst4, zeros_n, out, dst_v, ones_v, acc, sem):
    cid = lax.axis_index("c")
    sid = lax.axis_index("s")
    wid = sid * 2 + cid

    for i in range(CHUNK // 16):
        ones_v[pl.ds(i * 16, 16)] = jnp.ones((16,), jnp.float32)

    pltpu.sync_copy(zeros_n.at[pl.ds(sid * ROWS_PER_TILE, ROWS_PER_TILE)],
                    acc.at[pl.ds(sid * ROWS_PER_TILE, ROWS_PER_TILE)])
    pltpu.sync_copy(dst4.at[wid], dst_v)
    plsc.subcore_barrier()

    def chunk(c, carry):
        pltpu.sync_copy(ones_v, acc.at[dst_v.at[c].at[0]], add=True)
        return carry

    lax.fori_loop(0, NCHUNK, chunk, 0)
    plsc.subcore_barrier()

    pltpu.sync_copy(acc.at[pl.ds(sid * ROWS_PER_TILE, ROWS_PER_TILE)],
                    out.at[cid].at[pl.ds(sid * ROWS_PER_TILE, ROWS_PER_TILE)])


def _make_deg_kernel():
    return pl.kernel(
        _deg_body,
        out_type=jax.ShapeDtypeStruct((2, NPAD), jnp.float32),
        mesh=plsc.VectorSubcoreMesh(core_axis_name="c", subcore_axis_name="s"),
        compiler_params=_SC_PARAMS,
        scratch_types=[
            pltpu.VMEM((NCHUNK, 1, CHUNK), jnp.int32),
            pltpu.VMEM((CHUNK,), jnp.float32),
            pltpu.VMEM_SHARED((NPAD,), jnp.float32),
            pltpu.SemaphoreType.DMA,
        ],
    )


def _agg_body(src3, dst4, g, zeros, out, src_v, db0, db1, rows0, rows1, acc,
              sem0, sem1, semd0, semd1):
    cid = lax.axis_index("c")
    sid = lax.axis_index("s")
    wid = sid * 2 + cid
    rpt = ROWS_PER_TILE
    dst2 = dst4.at[wid]

    pltpu.sync_copy(zeros.at[pl.ds(sid * rpt, rpt)],
                    acc.at[pl.ds(sid * rpt, rpt)])
    pltpu.sync_copy(src3.at[wid], src_v)
    plsc.subcore_barrier()

    pltpu.async_copy(g.at[src_v.at[0]], rows0, sem0)
    pltpu.async_copy(dst2.at[0], db0, semd0)

    def pair(i, carry):
        c0 = 2 * i
        pltpu.async_copy(g.at[src_v.at[c0 + 1]], rows1, sem1)
        pltpu.async_copy(dst2.at[c0 + 1], db1, semd1)
        pltpu.make_async_copy(g.at[src_v.at[c0]], rows0, sem0).wait()
        pltpu.make_async_copy(dst2.at[c0], db0, semd0).wait()
        pltpu.sync_copy(rows0, acc.at[db0.at[0]], add=True)
        pltpu.async_copy(g.at[src_v.at[c0 + 2]], rows0, sem0)
        pltpu.async_copy(dst2.at[c0 + 2], db0, semd0)
        pltpu.make_async_copy(g.at[src_v.at[c0 + 1]], rows1, sem1).wait()
        pltpu.make_async_copy(dst2.at[c0 + 1], db1, semd1).wait()
        pltpu.sync_copy(rows1, acc.at[db1.at[0]], add=True)
        return carry

    lax.fori_loop(0, (NCHUNK - 1) // 2, pair, 0)
    pltpu.make_async_copy(g.at[src_v.at[NCHUNK - 1]], rows0, sem0).wait()
    pltpu.make_async_copy(dst2.at[NCHUNK - 1], db0, semd0).wait()
    pltpu.sync_copy(rows0, acc.at[db0.at[0]], add=True)
    plsc.subcore_barrier()

    pltpu.sync_copy(acc.at[pl.ds(sid * rpt, rpt)],
                    out.at[cid].at[pl.ds(sid * rpt, rpt)])


def _make_agg_kernel(feat):
    return pl.kernel(
        _agg_body,
        out_type=jax.ShapeDtypeStruct((2, NPAD, feat), jnp.float32),
        mesh=plsc.VectorSubcoreMesh(core_axis_name="c", subcore_axis_name="s"),
        compiler_params=_SC_PARAMS,
        scratch_types=[
            pltpu.VMEM((NCHUNK, CHUNK), jnp.int32),
            pltpu.VMEM((1, CHUNK), jnp.int32),
            pltpu.VMEM((1, CHUNK), jnp.int32),
            pltpu.VMEM((CHUNK, feat), jnp.float32),
            pltpu.VMEM((CHUNK, feat), jnp.float32),
            pltpu.VMEM_SHARED((NPAD, feat), jnp.float32),
            pltpu.SemaphoreType.DMA,
            pltpu.SemaphoreType.DMA,
            pltpu.SemaphoreType.DMA,
            pltpu.SemaphoreType.DMA,
        ],
    )


def _tc_a_body(x_ref, w1_ref, d0_ref, d1_ref, g1_ref, dinv_ref):
    dinv = lax.rsqrt(d0_ref[...] + d1_ref[...] + 1.0)
    h = jnp.dot(x_ref[...], w1_ref[...], preferred_element_type=jnp.float32)
    g1_ref[...] = h * dinv
    dinv_ref[...] = dinv


def _tc_a(x, w1, d0, d1):
    return pl.pallas_call(
        _tc_a_body,
        grid=(NBLKA,),
        in_specs=[
            pl.BlockSpec((BLKA, 128), lambda i: (i, 0)),
            pl.BlockSpec((128, 32), lambda i: (0, 0)),
            pl.BlockSpec((BLKA, 1), lambda i: (i, 0)),
            pl.BlockSpec((BLKA, 1), lambda i: (i, 0)),
        ],
        out_specs=[
            pl.BlockSpec((BLKA, 32), lambda i: (i, 0)),
            pl.BlockSpec((BLKA, 1), lambda i: (i, 0)),
        ],
        out_shape=[
            jax.ShapeDtypeStruct((N_NODES, 32), jnp.float32),
            jax.ShapeDtypeStruct((N_NODES, 1), jnp.float32),
        ],
    )(x, w1, d0, d1)


def _tc_b_body(a0_ref, a1_ref, g1_ref, dinv_ref, b1_ref, w2_ref, g2_ref):
    dinv = dinv_ref[...]
    o1 = ((a0_ref[...] + a1_ref[...] + g1_ref[...]) * dinv + b1_ref[...])
    o1 = jnp.maximum(o1, 0.0)
    h2 = jnp.dot(o1, w2_ref[...], preferred_element_type=jnp.float32)
    g2_ref[...] = h2 * dinv


def _tc_b(a0, a1, g1, dinv, b1, w2):
    return pl.pallas_call(
        _tc_b_body,
        grid=(NBLK,),
        in_specs=[
            pl.BlockSpec((BLK, 32), lambda i: (i, 0)),
            pl.BlockSpec((BLK, 32), lambda i: (i, 0)),
            pl.BlockSpec((BLK, 32), lambda i: (i, 0)),
            pl.BlockSpec((BLK, 1), lambda i: (i, 0)),
            pl.BlockSpec((1, 32), lambda i: (0, 0)),
            pl.BlockSpec((32, 64), lambda i: (0, 0)),
        ],
        out_specs=pl.BlockSpec((BLK, 64), lambda i: (i, 0)),
        out_shape=jax.ShapeDtypeStruct((N_NODES, 64), jnp.float32),
    )(a0, a1, g1, dinv, b1, w2)


def _tc_c_body(a0_ref, a1_ref, g2_ref, dinv_ref, b2_ref, out_ref):
    out_ref[...] = ((a0_ref[...] + a1_ref[...] + g2_ref[...])
                    * dinv_ref[...] + b2_ref[...])


def _tc_c(a0, a1, g2, dinv, b2):
    return pl.pallas_call(
        _tc_c_body,
        grid=(NBLK,),
        in_specs=[
            pl.BlockSpec((BLK, 64), lambda i: (i, 0)),
            pl.BlockSpec((BLK, 64), lambda i: (i, 0)),
            pl.BlockSpec((BLK, 64), lambda i: (i, 0)),
            pl.BlockSpec((BLK, 1), lambda i: (i, 0)),
            pl.BlockSpec((1, 64), lambda i: (0, 0)),
        ],
        out_specs=pl.BlockSpec((BLK, 64), lambda i: (i, 0)),
        out_shape=jax.ShapeDtypeStruct((N_NODES, 64), jnp.float32),
    )(a0, a1, g2, dinv, b2)


@jax.jit
def kernel(x, edge_index, W1, b1, W2, b2):
    src_e = edge_index[0].astype(jnp.int32)
    dst_e = edge_index[1].astype(jnp.int32)
    src3 = src_e.reshape(NW, NCHUNK, CHUNK)
    dst4 = dst_e.reshape(NW, NCHUNK, 1, CHUNK)

    zeros_n = jnp.zeros((NPAD,), jnp.float32)
    zeros32 = jnp.zeros((NPAD, 32), jnp.float32)
    zeros64 = jnp.zeros((NPAD, 64), jnp.float32)

    deg = _make_deg_kernel()(dst4, zeros_n)
    d0 = deg[0, :N_NODES].reshape(N_NODES, 1)
    d1 = deg[1, :N_NODES].reshape(N_NODES, 1)

    g1, dinv = _tc_a(x, W1, d0, d1)

    agg1 = _make_agg_kernel(32)(src3, dst4, g1, zeros32)
    g2 = _tc_b(agg1[0], agg1[1], g1, dinv, b1.reshape(1, 32), W2)

    agg2 = _make_agg_kernel(64)(src3, dst4, g2, zeros64)
    return _tc_c(agg2[0], agg2[1], g2, dinv, b2.reshape(1, 64))


# speedup vs baseline: 1.0962x; 1.0167x over previous
import jax
import jax.numpy as jnp
from jax import lax
from jax.experimental import pallas as pl
from jax.experimental.pallas import tpu as pltpu
from jax.experimental.pallas import tpu_sc as plsc

N_NODES = 10000
NPAD = 10240
N_EDGES = 320000
NW = 32
EDGES_PER_W = N_EDGES // NW
CHUNK = 80
NCHUNK = EDGES_PER_W // CHUNK
ROWS_PER_TILE = NPAD // 16
BLK = 1000
NBLK = N_NODES // BLK
BLKA = 2000
NBLKA = N_NODES // BLKA

_SC_PARAMS = pltpu.CompilerParams(use_tc_tiling_on_sc=False,
                                  skip_device_barrier=True)


def _deg_body(dst4, out, dst_v, ones_v, zbuf, acc, sem):
    cid = lax.axis_index("c")
    sid = lax.axis_index("s")
    wid = sid * 2 + cid

    for i in range(CHUNK // 16):
        ones_v[pl.ds(i * 16, 16)] = jnp.ones((16,), jnp.float32)
        zbuf[pl.ds(i * 16, 16)] = jnp.zeros((16,), jnp.float32)

    for k in range(ROWS_PER_TILE // CHUNK):
        pltpu.sync_copy(zbuf,
                        acc.at[pl.ds(sid * ROWS_PER_TILE + k * CHUNK, CHUNK)])
    pltpu.sync_copy(dst4.at[wid], dst_v)
    plsc.subcore_barrier()

    def chunk(c, carry):
        pltpu.sync_copy(ones_v, acc.at[dst_v.at[c].at[0]], add=True)
        return carry

    lax.fori_loop(0, NCHUNK, chunk, 0)
    plsc.subcore_barrier()

    pltpu.sync_copy(acc.at[pl.ds(sid * ROWS_PER_TILE, ROWS_PER_TILE)],
                    out.at[cid].at[pl.ds(sid * ROWS_PER_TILE, ROWS_PER_TILE)])


def _make_deg_kernel():
    return pl.kernel(
        _deg_body,
        out_type=jax.ShapeDtypeStruct((2, NPAD), jnp.float32),
        mesh=plsc.VectorSubcoreMesh(core_axis_name="c", subcore_axis_name="s"),
        compiler_params=_SC_PARAMS,
        scratch_types=[
            pltpu.VMEM((NCHUNK, 1, CHUNK), jnp.int32),
            pltpu.VMEM((CHUNK,), jnp.float32),
            pltpu.VMEM((CHUNK,), jnp.float32),
            pltpu.VMEM_SHARED((NPAD,), jnp.float32),
            pltpu.SemaphoreType.DMA,
        ],
    )


def _agg_body(src3, dst4, g, out, src_v, db0, db1, rows0, rows1, acc,
              sem0, sem1, semd0, semd1):
    cid = lax.axis_index("c")
    sid = lax.axis_index("s")
    wid = sid * 2 + cid
    rpt = ROWS_PER_TILE
    feat = rows0.shape[1]
    dst2 = dst4.at[wid]

    for r in range(CHUNK):
        for j in range(feat // 16):
            rows0[r, pl.ds(j * 16, 16)] = jnp.zeros((16,), jnp.float32)
    for k in range(rpt // CHUNK):
        pltpu.sync_copy(rows0, acc.at[pl.ds(sid * rpt + k * CHUNK, CHUNK)])
    pltpu.sync_copy(src3.at[wid], src_v)
    plsc.subcore_barrier()

    pltpu.async_copy(g.at[src_v.at[0]], rows0, sem0)
    pltpu.async_copy(dst2.at[0], db0, semd0)

    def pair(i, carry):
        c0 = 2 * i
        pltpu.async_copy(g.at[src_v.at[c0 + 1]], rows1, sem1)
        pltpu.async_copy(dst2.at[c0 + 1], db1, semd1)
        pltpu.make_async_copy(g.at[src_v.at[c0]], rows0, sem0).wait()
        pltpu.make_async_copy(dst2.at[c0], db0, semd0).wait()
        pltpu.sync_copy(rows0, acc.at[db0.at[0]], add=True)
        pltpu.async_copy(g.at[src_v.at[c0 + 2]], rows0, sem0)
        pltpu.async_copy(dst2.at[c0 + 2], db0, semd0)
        pltpu.make_async_copy(g.at[src_v.at[c0 + 1]], rows1, sem1).wait()
        pltpu.make_async_copy(dst2.at[c0 + 1], db1, semd1).wait()
        pltpu.sync_copy(rows1, acc.at[db1.at[0]], add=True)
        return carry

    lax.fori_loop(0, (NCHUNK - 1) // 2, pair, 0)
    pltpu.make_async_copy(g.at[src_v.at[NCHUNK - 1]], rows0, sem0).wait()
    pltpu.make_async_copy(dst2.at[NCHUNK - 1], db0, semd0).wait()
    pltpu.sync_copy(rows0, acc.at[db0.at[0]], add=True)
    plsc.subcore_barrier()

    pltpu.sync_copy(acc.at[pl.ds(sid * rpt, rpt)],
                    out.at[cid].at[pl.ds(sid * rpt, rpt)])


def _make_agg_kernel(feat):
    return pl.kernel(
        _agg_body,
        out_type=jax.ShapeDtypeStruct((2, NPAD, feat), jnp.float32),
        mesh=plsc.VectorSubcoreMesh(core_axis_name="c", subcore_axis_name="s"),
        compiler_params=_SC_PARAMS,
        scratch_types=[
            pltpu.VMEM((NCHUNK, CHUNK), jnp.int32),
            pltpu.VMEM((1, CHUNK), jnp.int32),
            pltpu.VMEM((1, CHUNK), jnp.int32),
            pltpu.VMEM((CHUNK, feat), jnp.float32),
            pltpu.VMEM((CHUNK, feat), jnp.float32),
            pltpu.VMEM_SHARED((NPAD, feat), jnp.float32),
            pltpu.SemaphoreType.DMA,
            pltpu.SemaphoreType.DMA,
            pltpu.SemaphoreType.DMA,
            pltpu.SemaphoreType.DMA,
        ],
    )


def _tc_a_body(x_ref, w1_ref, d0_ref, d1_ref, g1_ref, dinv_ref):
    dinv = lax.rsqrt(d0_ref[...] + d1_ref[...] + 1.0)
    h = jnp.dot(x_ref[...], w1_ref[...], preferred_element_type=jnp.float32)
    g1_ref[...] = h * dinv
    dinv_ref[...] = dinv


def _tc_a(x, w1, d0, d1):
    return pl.pallas_call(
        _tc_a_body,
        grid=(NBLKA,),
        in_specs=[
            pl.BlockSpec((BLKA, 128), lambda i: (i, 0)),
            pl.BlockSpec((128, 32), lambda i: (0, 0)),
            pl.BlockSpec((BLKA, 1), lambda i: (i, 0)),
            pl.BlockSpec((BLKA, 1), lambda i: (i, 0)),
        ],
        out_specs=[
            pl.BlockSpec((BLKA, 32), lambda i: (i, 0)),
            pl.BlockSpec((BLKA, 1), lambda i: (i, 0)),
        ],
        out_shape=[
            jax.ShapeDtypeStruct((N_NODES, 32), jnp.float32),
            jax.ShapeDtypeStruct((N_NODES, 1), jnp.float32),
        ],
    )(x, w1, d0, d1)


def _tc_b_body(a0_ref, a1_ref, g1_ref, dinv_ref, b1_ref, w2_ref, g2_ref):
    dinv = dinv_ref[...]
    o1 = ((a0_ref[...] + a1_ref[...] + g1_ref[...]) * dinv + b1_ref[...])
    o1 = jnp.maximum(o1, 0.0)
    h2 = jnp.dot(o1, w2_ref[...], preferred_element_type=jnp.float32)
    g2_ref[...] = h2 * dinv


def _tc_b(a0, a1, g1, dinv, b1, w2):
    return pl.pallas_call(
        _tc_b_body,
        grid=(NBLK,),
        in_specs=[
            pl.BlockSpec((BLK, 32), lambda i: (i, 0)),
            pl.BlockSpec((BLK, 32), lambda i: (i, 0)),
            pl.BlockSpec((BLK, 32), lambda i: (i, 0)),
            pl.BlockSpec((BLK, 1), lambda i: (i, 0)),
            pl.BlockSpec((1, 32), lambda i: (0, 0)),
            pl.BlockSpec((32, 64), lambda i: (0, 0)),
        ],
        out_specs=pl.BlockSpec((BLK, 64), lambda i: (i, 0)),
        out_shape=jax.ShapeDtypeStruct((N_NODES, 64), jnp.float32),
    )(a0, a1, g1, dinv, b1, w2)


def _tc_c_body(a0_ref, a1_ref, g2_ref, dinv_ref, b2_ref, out_ref):
    out_ref[...] = ((a0_ref[...] + a1_ref[...] + g2_ref[...])
                    * dinv_ref[...] + b2_ref[...])


def _tc_c(a0, a1, g2, dinv, b2):
    return pl.pallas_call(
        _tc_c_body,
        grid=(NBLK,),
        in_specs=[
            pl.BlockSpec((BLK, 64), lambda i: (i, 0)),
            pl.BlockSpec((BLK, 64), lambda i: (i, 0)),
            pl.BlockSpec((BLK, 64), lambda i: (i, 0)),
            pl.BlockSpec((BLK, 1), lambda i: (i, 0)),
            pl.BlockSpec((1, 64), lambda i: (0, 0)),
        ],
        out_specs=pl.BlockSpec((BLK, 64), lambda i: (i, 0)),
        out_shape=jax.ShapeDtypeStruct((N_NODES, 64), jnp.float32),
    )(a0, a1, g2, dinv, b2)


@jax.jit
def kernel(x, edge_index, W1, b1, W2, b2):
    src_e = edge_index[0].astype(jnp.int32)
    dst_e = edge_index[1].astype(jnp.int32)
    src3 = src_e.reshape(NW, NCHUNK, CHUNK)
    dst4 = dst_e.reshape(NW, NCHUNK, 1, CHUNK)

    deg = _make_deg_kernel()(dst4)
    d0 = deg[0, :N_NODES].reshape(N_NODES, 1)
    d1 = deg[1, :N_NODES].reshape(N_NODES, 1)

    g1, dinv = _tc_a(x, W1, d0, d1)

    agg1 = _make_agg_kernel(32)(src3, dst4, g1)
    g2 = _tc_b(agg1[0], agg1[1], g1, dinv, b1.reshape(1, 32), W2)

    agg2 = _make_agg_kernel(64)(src3, dst4, g2)
    return _tc_c(agg2[0], agg2[1], g2, dinv, b2.reshape(1, 64))


# speedup vs baseline: 1.1915x; 1.0869x over previous
import jax
import jax.numpy as jnp
from jax import lax
from jax.experimental import pallas as pl
from jax.experimental.pallas import tpu as pltpu
from jax.experimental.pallas import tpu_sc as plsc

N_NODES = 10000
NPAD = 10240
N_EDGES = 320000
NW = 32
EDGES_PER_W = N_EDGES // NW
CHUNK = 80
NCHUNK = EDGES_PER_W // CHUNK
ROWS_PER_TILE = NPAD // 16
BLK = 1000
NBLK = N_NODES // BLK
BLKA = 2000
NBLKA = N_NODES // BLKA

_SC_PARAMS = pltpu.CompilerParams(use_tc_tiling_on_sc=False,
                                  skip_device_barrier=True)


def _deg_body(dst4, out, dst_v, ones_v, zbuf, acc, sem):
    cid = lax.axis_index("c")
    sid = lax.axis_index("s")
    wid = sid * 2 + cid

    for i in range(CHUNK // 16):
        ones_v[pl.ds(i * 16, 16)] = jnp.ones((16,), jnp.float32)
        zbuf[pl.ds(i * 16, 16)] = jnp.zeros((16,), jnp.float32)

    for k in range(ROWS_PER_TILE // CHUNK):
        pltpu.sync_copy(zbuf,
                        acc.at[pl.ds(sid * ROWS_PER_TILE + k * CHUNK, CHUNK)])
    pltpu.sync_copy(dst4.at[wid], dst_v)
    plsc.subcore_barrier()

    def chunk(c, carry):
        pltpu.sync_copy(ones_v, acc.at[dst_v.at[c].at[0]], add=True)
        return carry

    lax.fori_loop(0, NCHUNK, chunk, 0)
    plsc.subcore_barrier()

    pltpu.sync_copy(acc.at[pl.ds(sid * ROWS_PER_TILE, ROWS_PER_TILE)],
                    out.at[cid].at[pl.ds(sid * ROWS_PER_TILE, ROWS_PER_TILE)])


def _make_deg_kernel():
    return pl.kernel(
        _deg_body,
        out_type=jax.ShapeDtypeStruct((2, NPAD), jnp.float32),
        mesh=plsc.VectorSubcoreMesh(core_axis_name="c", subcore_axis_name="s"),
        compiler_params=_SC_PARAMS,
        scratch_types=[
            pltpu.VMEM((NCHUNK, 1, CHUNK), jnp.int32),
            pltpu.VMEM((CHUNK,), jnp.float32),
            pltpu.VMEM((CHUNK,), jnp.float32),
            pltpu.VMEM_SHARED((NPAD,), jnp.float32),
            pltpu.SemaphoreType.DMA,
        ],
    )


def _agg_body(src3, dst4, g, out, src_v, db0, db1, db2, rows0, rows1, rows2,
              acc, g0, g1s, g2s, i0, i1, i2, s0, s1, s2):
    cid = lax.axis_index("c")
    sid = lax.axis_index("s")
    wid = sid * 2 + cid
    rpt = ROWS_PER_TILE
    feat = rows0.shape[1]
    dst2 = dst4.at[wid]
    last = NCHUNK - 1

    for r in range(CHUNK):
        for j in range(feat // 16):
            rows0[r, pl.ds(j * 16, 16)] = jnp.zeros((16,), jnp.float32)
    for k in range(rpt // CHUNK):
        pltpu.sync_copy(rows0, acc.at[pl.ds(sid * rpt + k * CHUNK, CHUNK)])
    pltpu.sync_copy(src3.at[wid], src_v)
    plsc.subcore_barrier()

    pltpu.async_copy(g.at[src_v.at[0]], rows0, g0)
    pltpu.async_copy(dst2.at[0], db0, i0)
    pltpu.async_copy(g.at[src_v.at[1]], rows1, g1s)
    pltpu.async_copy(dst2.at[1], db1, i1)
    pltpu.async_copy(g.at[src_v.at[2]], rows2, g2s)
    pltpu.async_copy(dst2.at[2], db2, i2)

    def iter3(i, carry):
        c0 = 3 * i
        pltpu.make_async_copy(g.at[src_v.at[c0]], rows0, g0).wait()
        pltpu.make_async_copy(dst2.at[c0], db0, i0).wait()
        pltpu.async_copy(rows0, acc.at[db0.at[0]], s0, add=True)
        pltpu.make_async_copy(g.at[src_v.at[c0]], rows1, g1s).wait()
        pltpu.make_async_copy(dst2.at[c0], db1, i1).wait()
        pltpu.async_copy(rows1, acc.at[db1.at[0]], s1, add=True)
        pltpu.make_async_copy(g.at[src_v.at[c0]], rows2, g2s).wait()
        pltpu.make_async_copy(dst2.at[c0], db2, i2).wait()
        pltpu.async_copy(rows2, acc.at[db2.at[0]], s2, add=True)
        n1 = c0 + 3
        n2 = jnp.minimum(c0 + 4, last)
        n3 = jnp.minimum(c0 + 5, last)
        pltpu.make_async_copy(rows0, acc.at[db0.at[0]], s0).wait()
        pltpu.async_copy(g.at[src_v.at[n1]], rows0, g0)
        pltpu.async_copy(dst2.at[n1], db0, i0)
        pltpu.make_async_copy(rows1, acc.at[db1.at[0]], s1).wait()
        pltpu.async_copy(g.at[src_v.at[n2]], rows1, g1s)
        pltpu.async_copy(dst2.at[n2], db1, i1)
        pltpu.make_async_copy(rows2, acc.at[db2.at[0]], s2).wait()
        pltpu.async_copy(g.at[src_v.at[n3]], rows2, g2s)
        pltpu.async_copy(dst2.at[n3], db2, i2)
        return carry

    lax.fori_loop(0, (NCHUNK - 2) // 3, iter3, 0)
    pltpu.make_async_copy(g.at[src_v.at[0]], rows0, g0).wait()
    pltpu.make_async_copy(dst2.at[0], db0, i0).wait()
    pltpu.async_copy(rows0, acc.at[db0.at[0]], s0, add=True)
    pltpu.make_async_copy(g.at[src_v.at[0]], rows1, g1s).wait()
    pltpu.make_async_copy(dst2.at[0], db1, i1).wait()
    pltpu.async_copy(rows1, acc.at[db1.at[0]], s1, add=True)
    pltpu.make_async_copy(g.at[src_v.at[0]], rows2, g2s).wait()
    pltpu.make_async_copy(dst2.at[0], db2, i2).wait()
    pltpu.make_async_copy(rows0, acc.at[db0.at[0]], s0).wait()
    pltpu.make_async_copy(rows1, acc.at[db1.at[0]], s1).wait()
    plsc.subcore_barrier()

    pltpu.sync_copy(acc.at[pl.ds(sid * rpt, rpt)],
                    out.at[cid].at[pl.ds(sid * rpt, rpt)])


def _make_agg_kernel(feat):
    return pl.kernel(
        _agg_body,
        out_type=jax.ShapeDtypeStruct((2, NPAD, feat), jnp.float32),
        mesh=plsc.VectorSubcoreMesh(core_axis_name="c", subcore_axis_name="s"),
        compiler_params=_SC_PARAMS,
        scratch_types=[
            pltpu.VMEM((NCHUNK, CHUNK), jnp.int32),
            pltpu.VMEM((1, CHUNK), jnp.int32),
            pltpu.VMEM((1, CHUNK), jnp.int32),
            pltpu.VMEM((1, CHUNK), jnp.int32),
            pltpu.VMEM((CHUNK, feat), jnp.float32),
            pltpu.VMEM((CHUNK, feat), jnp.float32),
            pltpu.VMEM((CHUNK, feat), jnp.float32),
            pltpu.VMEM_SHARED((NPAD, feat), jnp.float32),
        ] + [pltpu.SemaphoreType.DMA] * 9,
    )


def _tc_a_body(x_ref, w1_ref, d0_ref, d1_ref, g1_ref, dinv_ref):
    dinv = lax.rsqrt(d0_ref[...] + d1_ref[...] + 1.0)
    h = jnp.dot(x_ref[...], w1_ref[...], preferred_element_type=jnp.float32)
    g1_ref[...] = h * dinv
    dinv_ref[...] = dinv


def _tc_a(x, w1, d0, d1):
    return pl.pallas_call(
        _tc_a_body,
        grid=(NBLKA,),
        in_specs=[
            pl.BlockSpec((BLKA, 128), lambda i: (i, 0)),
            pl.BlockSpec((128, 32), lambda i: (0, 0)),
            pl.BlockSpec((BLKA, 1), lambda i: (i, 0)),
            pl.BlockSpec((BLKA, 1), lambda i: (i, 0)),
        ],
        out_specs=[
            pl.BlockSpec((BLKA, 32), lambda i: (i, 0)),
            pl.BlockSpec((BLKA, 1), lambda i: (i, 0)),
        ],
        out_shape=[
            jax.ShapeDtypeStruct((N_NODES, 32), jnp.float32),
            jax.ShapeDtypeStruct((N_NODES, 1), jnp.float32),
        ],
    )(x, w1, d0, d1)


def _tc_b_body(a0_ref, a1_ref, g1_ref, dinv_ref, b1_ref, w2_ref, g2_ref):
    dinv = dinv_ref[...]
    o1 = ((a0_ref[...] + a1_ref[...] + g1_ref[...]) * dinv + b1_ref[...])
    o1 = jnp.maximum(o1, 0.0)
    h2 = jnp.dot(o1, w2_ref[...], preferred_element_type=jnp.float32)
    g2_ref[...] = h2 * dinv


def _tc_b(a0, a1, g1, dinv, b1, w2):
    return pl.pallas_call(
        _tc_b_body,
        grid=(NBLK,),
        in_specs=[
            pl.BlockSpec((BLK, 32), lambda i: (i, 0)),
            pl.BlockSpec((BLK, 32), lambda i: (i, 0)),
            pl.BlockSpec((BLK, 32), lambda i: (i, 0)),
            pl.BlockSpec((BLK, 1), lambda i: (i, 0)),
            pl.BlockSpec((1, 32), lambda i: (0, 0)),
            pl.BlockSpec((32, 64), lambda i: (0, 0)),
        ],
        out_specs=pl.BlockSpec((BLK, 64), lambda i: (i, 0)),
        out_shape=jax.ShapeDtypeStruct((N_NODES, 64), jnp.float32),
    )(a0, a1, g1, dinv, b1, w2)


def _tc_c_body(a0_ref, a1_ref, g2_ref, dinv_ref, b2_ref, out_ref):
    out_ref[...] = ((a0_ref[...] + a1_ref[...] + g2_ref[...])
                    * dinv_ref[...] + b2_ref[...])


def _tc_c(a0, a1, g2, dinv, b2):
    return pl.pallas_call(
        _tc_c_body,
        grid=(NBLK,),
        in_specs=[
            pl.BlockSpec((BLK, 64), lambda i: (i, 0)),
            pl.BlockSpec((BLK, 64), lambda i: (i, 0)),
            pl.BlockSpec((BLK, 64), lambda i: (i, 0)),
            pl.BlockSpec((BLK, 1), lambda i: (i, 0)),
            pl.BlockSpec((1, 64), lambda i: (0, 0)),
        ],
        out_specs=pl.BlockSpec((BLK, 64), lambda i: (i, 0)),
        out_shape=jax.ShapeDtypeStruct((N_NODES, 64), jnp.float32),
    )(a0, a1, g2, dinv, b2)


@jax.jit
def kernel(x, edge_index, W1, b1, W2, b2):
    src_e = edge_index[0].astype(jnp.int32)
    dst_e = edge_index[1].astype(jnp.int32)
    src3 = src_e.reshape(NW, NCHUNK, CHUNK)
    dst4 = dst_e.reshape(NW, NCHUNK, 1, CHUNK)

    deg = _make_deg_kernel()(dst4)
    d0 = deg[0, :N_NODES].reshape(N_NODES, 1)
    d1 = deg[1, :N_NODES].reshape(N_NODES, 1)

    g1, dinv = _tc_a(x, W1, d0, d1)

    agg1 = _make_agg_kernel(32)(src3, dst4, g1)
    g2 = _tc_b(agg1[0], agg1[1], g1, dinv, b1.reshape(1, 32), W2)

    agg2 = _make_agg_kernel(64)(src3, dst4, g2)
    return _tc_c(agg2[0], agg2[1], g2, dinv, b2.reshape(1, 64))


# speedup vs baseline: 1.2297x; 1.0320x over previous
import jax
import jax.numpy as jnp
from jax import lax
from jax.experimental import pallas as pl
from jax.experimental.pallas import tpu as pltpu
from jax.experimental.pallas import tpu_sc as plsc

N_NODES = 10000
NPAD = 10240
N_EDGES = 320000
NW = 32
EDGES_PER_W = N_EDGES // NW
CHUNK = 80
NCHUNK = EDGES_PER_W // CHUNK
ROWS_PER_TILE = NPAD // 16
BLK = 1000
NBLK = N_NODES // BLK
BLKA = 2000
NBLKA = N_NODES // BLKA

_SC_PARAMS = pltpu.CompilerParams(use_tc_tiling_on_sc=False,
                                  skip_device_barrier=True)


def _deg_body(dst4, out, dst_v, ones_v, zbuf, acc, sem):
    cid = lax.axis_index("c")
    sid = lax.axis_index("s")
    wid = sid * 2 + cid

    for i in range(CHUNK // 16):
        ones_v[pl.ds(i * 16, 16)] = jnp.ones((16,), jnp.float32)
        zbuf[pl.ds(i * 16, 16)] = jnp.zeros((16,), jnp.float32)

    for k in range(ROWS_PER_TILE // CHUNK):
        pltpu.sync_copy(zbuf,
                        acc.at[pl.ds(sid * ROWS_PER_TILE + k * CHUNK, CHUNK)])
    pltpu.sync_copy(dst4.at[wid], dst_v)
    plsc.subcore_barrier()

    def chunk(c, carry):
        pltpu.async_copy(ones_v, acc.at[dst_v.at[c].at[0]], sem, add=True)
        return carry

    lax.fori_loop(0, NCHUNK, chunk, 0)

    def drain(c, carry):
        pltpu.make_async_copy(ones_v, acc.at[dst_v.at[0].at[0]], sem).wait()
        return carry

    lax.fori_loop(0, NCHUNK, drain, 0)
    plsc.subcore_barrier()

    pltpu.sync_copy(acc.at[pl.ds(sid * ROWS_PER_TILE, ROWS_PER_TILE)],
                    out.at[cid].at[pl.ds(sid * ROWS_PER_TILE, ROWS_PER_TILE)])


def _make_deg_kernel():
    return pl.kernel(
        _deg_body,
        out_type=jax.ShapeDtypeStruct((2, NPAD), jnp.float32),
        mesh=plsc.VectorSubcoreMesh(core_axis_name="c", subcore_axis_name="s"),
        compiler_params=_SC_PARAMS,
        scratch_types=[
            pltpu.VMEM((NCHUNK, 1, CHUNK), jnp.int32),
            pltpu.VMEM((CHUNK,), jnp.float32),
            pltpu.VMEM((CHUNK,), jnp.float32),
            pltpu.VMEM_SHARED((NPAD,), jnp.float32),
            pltpu.SemaphoreType.DMA,
        ],
    )


def _agg_body(src3, dst4, g, out, src_v, db0, db1, db2, rows0, rows1, rows2,
              acc, g0, g1s, g2s, i0, i1, i2, s0, s1, s2):
    cid = lax.axis_index("c")
    sid = lax.axis_index("s")
    wid = sid * 2 + cid
    rpt = ROWS_PER_TILE
    feat = rows0.shape[1]
    dst2 = dst4.at[wid]
    last = NCHUNK - 1

    for r in range(CHUNK):
        for j in range(feat // 16):
            rows0[r, pl.ds(j * 16, 16)] = jnp.zeros((16,), jnp.float32)
    for k in range(rpt // CHUNK):
        pltpu.sync_copy(rows0, acc.at[pl.ds(sid * rpt + k * CHUNK, CHUNK)])
    pltpu.sync_copy(src3.at[wid], src_v)
    plsc.subcore_barrier()

    pltpu.async_copy(g.at[src_v.at[0]], rows0, g0)
    pltpu.async_copy(dst2.at[0], db0, i0)
    pltpu.async_copy(g.at[src_v.at[1]], rows1, g1s)
    pltpu.async_copy(dst2.at[1], db1, i1)
    pltpu.async_copy(g.at[src_v.at[2]], rows2, g2s)
    pltpu.async_copy(dst2.at[2], db2, i2)

    def iter3(i, carry):
        c0 = 3 * i
        pltpu.make_async_copy(g.at[src_v.at[c0]], rows0, g0).wait()
        pltpu.make_async_copy(dst2.at[c0], db0, i0).wait()
        pltpu.async_copy(rows0, acc.at[db0.at[0]], s0, add=True)
        pltpu.make_async_copy(g.at[src_v.at[c0]], rows1, g1s).wait()
        pltpu.make_async_copy(dst2.at[c0], db1, i1).wait()
        pltpu.async_copy(rows1, acc.at[db1.at[0]], s1, add=True)
        pltpu.make_async_copy(g.at[src_v.at[c0]], rows2, g2s).wait()
        pltpu.make_async_copy(dst2.at[c0], db2, i2).wait()
        pltpu.async_copy(rows2, acc.at[db2.at[0]], s2, add=True)
        n1 = c0 + 3
        n2 = jnp.minimum(c0 + 4, last)
        n3 = jnp.minimum(c0 + 5, last)
        pltpu.make_async_copy(rows0, acc.at[db0.at[0]], s0).wait()
        pltpu.async_copy(g.at[src_v.at[n1]], rows0, g0)
        pltpu.async_copy(dst2.at[n1], db0, i0)
        pltpu.make_async_copy(rows1, acc.at[db1.at[0]], s1).wait()
        pltpu.async_copy(g.at[src_v.at[n2]], rows1, g1s)
        pltpu.async_copy(dst2.at[n2], db1, i1)
        pltpu.make_async_copy(rows2, acc.at[db2.at[0]], s2).wait()
        pltpu.async_copy(g.at[src_v.at[n3]], rows2, g2s)
        pltpu.async_copy(dst2.at[n3], db2, i2)
        return carry

    lax.fori_loop(0, (NCHUNK - 2) // 3, iter3, 0)
    pltpu.make_async_copy(g.at[src_v.at[0]], rows0, g0).wait()
    pltpu.make_async_copy(dst2.at[0], db0, i0).wait()
    pltpu.async_copy(rows0, acc.at[db0.at[0]], s0, add=True)
    pltpu.make_async_copy(g.at[src_v.at[0]], rows1, g1s).wait()
    pltpu.make_async_copy(dst2.at[0], db1, i1).wait()
    pltpu.async_copy(rows1, acc.at[db1.at[0]], s1, add=True)
    pltpu.make_async_copy(g.at[src_v.at[0]], rows2, g2s).wait()
    pltpu.make_async_copy(dst2.at[0], db2, i2).wait()
    pltpu.make_async_copy(rows0, acc.at[db0.at[0]], s0).wait()
    pltpu.make_async_copy(rows1, acc.at[db1.at[0]], s1).wait()
    plsc.subcore_barrier()

    pltpu.sync_copy(acc.at[pl.ds(sid * rpt, rpt)],
                    out.at[cid].at[pl.ds(sid * rpt, rpt)])


def _make_agg_kernel(feat):
    return pl.kernel(
        _agg_body,
        out_type=jax.ShapeDtypeStruct((2, NPAD, feat), jnp.float32),
        mesh=plsc.VectorSubcoreMesh(core_axis_name="c", subcore_axis_name="s"),
        compiler_params=_SC_PARAMS,
        scratch_types=[
            pltpu.VMEM((NCHUNK, CHUNK), jnp.int32),
            pltpu.VMEM((1, CHUNK), jnp.int32),
            pltpu.VMEM((1, CHUNK), jnp.int32),
            pltpu.VMEM((1, CHUNK), jnp.int32),
            pltpu.VMEM((CHUNK, feat), jnp.float32),
            pltpu.VMEM((CHUNK, feat), jnp.float32),
            pltpu.VMEM((CHUNK, feat), jnp.float32),
            pltpu.VMEM_SHARED((NPAD, feat), jnp.float32),
        ] + [pltpu.SemaphoreType.DMA] * 9,
    )


def _tc_a_body(x_ref, w1_ref, d0_ref, d1_ref, g1_ref, dinv_ref):
    dinv = lax.rsqrt(d0_ref[...] + d1_ref[...] + 1.0)
    h = jnp.dot(x_ref[...], w1_ref[...], preferred_element_type=jnp.float32)
    g1_ref[...] = h * dinv
    dinv_ref[...] = dinv


def _tc_a(x, w1, d0, d1):
    return pl.pallas_call(
        _tc_a_body,
        grid=(NBLKA,),
        in_specs=[
            pl.BlockSpec((BLKA, 128), lambda i: (i, 0)),
            pl.BlockSpec((128, 32), lambda i: (0, 0)),
            pl.BlockSpec((BLKA, 1), lambda i: (i, 0)),
            pl.BlockSpec((BLKA, 1), lambda i: (i, 0)),
        ],
        out_specs=[
            pl.BlockSpec((BLKA, 32), lambda i: (i, 0)),
            pl.BlockSpec((BLKA, 1), lambda i: (i, 0)),
        ],
        out_shape=[
            jax.ShapeDtypeStruct((N_NODES, 32), jnp.float32),
            jax.ShapeDtypeStruct((N_NODES, 1), jnp.float32),
        ],
    )(x, w1, d0, d1)


def _tc_b_body(a0_ref, a1_ref, g1_ref, dinv_ref, b1_ref, w2_ref, g2_ref):
    dinv = dinv_ref[...]
    o1 = ((a0_ref[...] + a1_ref[...] + g1_ref[...]) * dinv + b1_ref[...])
    o1 = jnp.maximum(o1, 0.0)
    h2 = jnp.dot(o1, w2_ref[...], preferred_element_type=jnp.float32)
    g2_ref[...] = h2 * dinv


def _tc_b(a0, a1, g1, dinv, b1, w2):
    return pl.pallas_call(
        _tc_b_body,
        grid=(NBLK,),
        in_specs=[
            pl.BlockSpec((BLK, 32), lambda i: (i, 0)),
            pl.BlockSpec((BLK, 32), lambda i: (i, 0)),
            pl.BlockSpec((BLK, 32), lambda i: (i, 0)),
            pl.BlockSpec((BLK, 1), lambda i: (i, 0)),
            pl.BlockSpec((1, 32), lambda i: (0, 0)),
            pl.BlockSpec((32, 64), lambda i: (0, 0)),
        ],
        out_specs=pl.BlockSpec((BLK, 64), lambda i: (i, 0)),
        out_shape=jax.ShapeDtypeStruct((N_NODES, 64), jnp.float32),
    )(a0, a1, g1, dinv, b1, w2)


def _tc_c_body(a0_ref, a1_ref, g2_ref, dinv_ref, b2_ref, out_ref):
    out_ref[...] = ((a0_ref[...] + a1_ref[...] + g2_ref[...])
                    * dinv_ref[...] + b2_ref[...])


def _tc_c(a0, a1, g2, dinv, b2):
    return pl.pallas_call(
        _tc_c_body,
        grid=(NBLK,),
        in_specs=[
            pl.BlockSpec((BLK, 64), lambda i: (i, 0)),
            pl.BlockSpec((BLK, 64), lambda i: (i, 0)),
            pl.BlockSpec((BLK, 64), lambda i: (i, 0)),
            pl.BlockSpec((BLK, 1), lambda i: (i, 0)),
            pl.BlockSpec((1, 64), lambda i: (0, 0)),
        ],
        out_specs=pl.BlockSpec((BLK, 64), lambda i: (i, 0)),
        out_shape=jax.ShapeDtypeStruct((N_NODES, 64), jnp.float32),
    )(a0, a1, g2, dinv, b2)


@jax.jit
def kernel(x, edge_index, W1, b1, W2, b2):
    src_e = edge_index[0].astype(jnp.int32)
    dst_e = edge_index[1].astype(jnp.int32)
    src3 = src_e.reshape(NW, NCHUNK, CHUNK)
    dst4 = dst_e.reshape(NW, NCHUNK, 1, CHUNK)

    deg = _make_deg_kernel()(dst4)
    d0 = deg[0, :N_NODES].reshape(N_NODES, 1)
    d1 = deg[1, :N_NODES].reshape(N_NODES, 1)

    g1, dinv = _tc_a(x, W1, d0, d1)

    agg1 = _make_agg_kernel(32)(src3, dst4, g1)
    g2 = _tc_b(agg1[0], agg1[1], g1, dinv, b1.reshape(1, 32), W2)

    agg2 = _make_agg_kernel(64)(src3, dst4, g2)
    return _tc_c(agg2[0], agg2[1], g2, dinv, b2.reshape(1, 64))


# speedup vs baseline: 1.3496x; 1.0975x over previous
import jax
import jax.numpy as jnp
from jax import lax
from jax.experimental import pallas as pl
from jax.experimental.pallas import tpu as pltpu
from jax.experimental.pallas import tpu_sc as plsc

N_NODES = 10000
NPAD = 10240
N_EDGES = 320000
NW = 32
EDGES_PER_W = N_EDGES // NW
CHUNK = 80
NCHUNK = EDGES_PER_W // CHUNK
ROWS_PER_TILE = NPAD // 16
BLK = 1000
NBLK = N_NODES // BLK
BLKA = 2000
NBLKA = N_NODES // BLKA

_SC_PARAMS = pltpu.CompilerParams(use_tc_tiling_on_sc=False,
                                  skip_device_barrier=True)


def _deg_body(dst4, out, dst_v, ones_v, zbuf, acc, sem):
    cid = lax.axis_index("c")
    sid = lax.axis_index("s")
    wid = sid * 2 + cid

    for i in range(CHUNK // 16):
        ones_v[pl.ds(i * 16, 16)] = jnp.ones((16,), jnp.float32)
        zbuf[pl.ds(i * 16, 16)] = jnp.zeros((16,), jnp.float32)

    for k in range(ROWS_PER_TILE // CHUNK):
        pltpu.sync_copy(zbuf,
                        acc.at[pl.ds(sid * ROWS_PER_TILE + k * CHUNK, CHUNK)])
    pltpu.sync_copy(dst4.at[wid], dst_v)
    plsc.subcore_barrier()

    def chunk(c, carry):
        pltpu.async_copy(ones_v, acc.at[dst_v.at[c].at[0]], sem, add=True)
        return carry

    lax.fori_loop(0, NCHUNK, chunk, 0)

    def drain(c, carry):
        pltpu.make_async_copy(ones_v, acc.at[dst_v.at[0].at[0]], sem).wait()
        return carry

    lax.fori_loop(0, NCHUNK, drain, 0)
    plsc.subcore_barrier()

    pltpu.sync_copy(acc.at[pl.ds(sid * ROWS_PER_TILE, ROWS_PER_TILE)],
                    out.at[cid].at[pl.ds(sid * ROWS_PER_TILE, ROWS_PER_TILE)])


def _make_deg_kernel():
    return pl.kernel(
        _deg_body,
        out_type=jax.ShapeDtypeStruct((2, NPAD), jnp.float32),
        mesh=plsc.VectorSubcoreMesh(core_axis_name="c", subcore_axis_name="s"),
        compiler_params=_SC_PARAMS,
        scratch_types=[
            pltpu.VMEM((NCHUNK, 1, CHUNK), jnp.int32),
            pltpu.VMEM((CHUNK,), jnp.float32),
            pltpu.VMEM((CHUNK,), jnp.float32),
            pltpu.VMEM_SHARED((NPAD,), jnp.float32),
            pltpu.SemaphoreType.DMA,
        ],
    )


def _agg_body(src3, dst4, g, out, src_v, dbs, rowss, acc, gsems, isems, ssems):
    cid = lax.axis_index("c")
    sid = lax.axis_index("s")
    wid = sid * 2 + cid
    rpt = ROWS_PER_TILE
    feat = rowss[0].shape[1]
    dst2 = dst4.at[wid]
    NB = 5

    r0 = rowss[0]
    for r in range(CHUNK):
        for j in range(feat // 16):
            r0[r, pl.ds(j * 16, 16)] = jnp.zeros((16,), jnp.float32)
    for k in range(rpt // CHUNK):
        pltpu.sync_copy(r0, acc.at[pl.ds(sid * rpt + k * CHUNK, CHUNK)])
    pltpu.sync_copy(src3.at[wid], src_v)
    plsc.subcore_barrier()

    for b in range(NB):
        pltpu.async_copy(g.at[src_v.at[b]], rowss[b], gsems[b])
        pltpu.async_copy(dst2.at[b], dbs[b], isems[b])

    def iter5(i, carry):
        c0 = 5 * i
        for b in range(NB):
            pltpu.make_async_copy(g.at[src_v.at[c0]], rowss[b],
                                  gsems[b]).wait()
            pltpu.make_async_copy(dst2.at[c0], dbs[b], isems[b]).wait()
            pltpu.async_copy(rowss[b], acc.at[dbs[b].at[0]], ssems[b],
                             add=True)
        for b in range(NB):
            pltpu.make_async_copy(rowss[b], acc.at[dbs[b].at[0]],
                                  ssems[b]).wait()
            pltpu.async_copy(g.at[src_v.at[c0 + NB + b]], rowss[b], gsems[b])
            pltpu.async_copy(dst2.at[c0 + NB + b], dbs[b], isems[b])
        return carry

    lax.fori_loop(0, NCHUNK // 5 - 1, iter5, 0)
    for b in range(NB):
        pltpu.make_async_copy(g.at[src_v.at[0]], rowss[b], gsems[b]).wait()
        pltpu.make_async_copy(dst2.at[0], dbs[b], isems[b]).wait()
        pltpu.async_copy(rowss[b], acc.at[dbs[b].at[0]], ssems[b], add=True)
    for b in range(NB):
        pltpu.make_async_copy(rowss[b], acc.at[dbs[b].at[0]], ssems[b]).wait()
    plsc.subcore_barrier()

    pltpu.sync_copy(acc.at[pl.ds(sid * rpt, rpt)],
                    out.at[cid].at[pl.ds(sid * rpt, rpt)])


def _make_agg_kernel(feat):
    return pl.kernel(
        _agg_body,
        out_type=jax.ShapeDtypeStruct((2, NPAD, feat), jnp.float32),
        mesh=plsc.VectorSubcoreMesh(core_axis_name="c", subcore_axis_name="s"),
        compiler_params=_SC_PARAMS,
        scratch_types=[
            pltpu.VMEM((NCHUNK, CHUNK), jnp.int32),
            [pltpu.VMEM((1, CHUNK), jnp.int32)] * 5,
            [pltpu.VMEM((CHUNK, feat), jnp.float32)] * 5,
            pltpu.VMEM_SHARED((NPAD, feat), jnp.float32),
            [pltpu.SemaphoreType.DMA] * 5,
            [pltpu.SemaphoreType.DMA] * 5,
            [pltpu.SemaphoreType.DMA] * 5,
        ],
    )


def _tc_a_body(x_ref, w1_ref, d0_ref, d1_ref, g1_ref, dinv_ref):
    dinv = lax.rsqrt(d0_ref[...] + d1_ref[...] + 1.0)
    h = jnp.dot(x_ref[...], w1_ref[...], preferred_element_type=jnp.float32)
    g1_ref[...] = h * dinv
    dinv_ref[...] = dinv


def _tc_a(x, w1, d0, d1):
    return pl.pallas_call(
        _tc_a_body,
        grid=(NBLKA,),
        in_specs=[
            pl.BlockSpec((BLKA, 128), lambda i: (i, 0)),
            pl.BlockSpec((128, 32), lambda i: (0, 0)),
            pl.BlockSpec((BLKA, 1), lambda i: (i, 0)),
            pl.BlockSpec((BLKA, 1), lambda i: (i, 0)),
        ],
        out_specs=[
            pl.BlockSpec((BLKA, 32), lambda i: (i, 0)),
            pl.BlockSpec((BLKA, 1), lambda i: (i, 0)),
        ],
        out_shape=[
            jax.ShapeDtypeStruct((N_NODES, 32), jnp.float32),
            jax.ShapeDtypeStruct((N_NODES, 1), jnp.float32),
        ],
    )(x, w1, d0, d1)


def _tc_b_body(a0_ref, a1_ref, g1_ref, dinv_ref, b1_ref, w2_ref, g2_ref):
    dinv = dinv_ref[...]
    o1 = ((a0_ref[...] + a1_ref[...] + g1_ref[...]) * dinv + b1_ref[...])
    o1 = jnp.maximum(o1, 0.0)
    h2 = jnp.dot(o1, w2_ref[...], preferred_element_type=jnp.float32)
    g2_ref[...] = h2 * dinv


def _tc_b(a0, a1, g1, dinv, b1, w2):
    return pl.pallas_call(
        _tc_b_body,
        grid=(NBLK,),
        in_specs=[
            pl.BlockSpec((BLK, 32), lambda i: (i, 0)),
            pl.BlockSpec((BLK, 32), lambda i: (i, 0)),
            pl.BlockSpec((BLK, 32), lambda i: (i, 0)),
            pl.BlockSpec((BLK, 1), lambda i: (i, 0)),
            pl.BlockSpec((1, 32), lambda i: (0, 0)),
            pl.BlockSpec((32, 64), lambda i: (0, 0)),
        ],
        out_specs=pl.BlockSpec((BLK, 64), lambda i: (i, 0)),
        out_shape=jax.ShapeDtypeStruct((N_NODES, 64), jnp.float32),
    )(a0, a1, g1, dinv, b1, w2)


def _tc_c_body(a0_ref, a1_ref, g2_ref, dinv_ref, b2_ref, out_ref):
    out_ref[...] = ((a0_ref[...] + a1_ref[...] + g2_ref[...])
                    * dinv_ref[...] + b2_ref[...])


def _tc_c(a0, a1, g2, dinv, b2):
    return pl.pallas_call(
        _tc_c_body,
        grid=(NBLK,),
        in_specs=[
            pl.BlockSpec((BLK, 64), lambda i: (i, 0)),
            pl.BlockSpec((BLK, 64), lambda i: (i, 0)),
            pl.BlockSpec((BLK, 64), lambda i: (i, 0)),
            pl.BlockSpec((BLK, 1), lambda i: (i, 0)),
            pl.BlockSpec((1, 64), lambda i: (0, 0)),
        ],
        out_specs=pl.BlockSpec((BLK, 64), lambda i: (i, 0)),
        out_shape=jax.ShapeDtypeStruct((N_NODES, 64), jnp.float32),
    )(a0, a1, g2, dinv, b2)


@jax.jit
def kernel(x, edge_index, W1, b1, W2, b2):
    src_e = edge_index[0].astype(jnp.int32)
    dst_e = edge_index[1].astype(jnp.int32)
    src3 = src_e.reshape(NW, NCHUNK, CHUNK)
    dst4 = dst_e.reshape(NW, NCHUNK, 1, CHUNK)

    deg = _make_deg_kernel()(dst4)
    d0 = deg[0, :N_NODES].reshape(N_NODES, 1)
    d1 = deg[1, :N_NODES].reshape(N_NODES, 1)

    g1, dinv = _tc_a(x, W1, d0, d1)

    agg1 = _make_agg_kernel(32)(src3, dst4, g1)
    g2 = _tc_b(agg1[0], agg1[1], g1, dinv, b1.reshape(1, 32), W2)

    agg2 = _make_agg_kernel(64)(src3, dst4, g2)
    return _tc_c(agg2[0], agg2[1], g2, dinv, b2.reshape(1, 64))
